# baseline (speedup 1.0000x reference)
import functools

import jax
import jax.numpy as jnp
from jax import lax
from jax.experimental import pallas as pl
from jax.experimental.pallas import tpu as pltpu
from jax.experimental.pallas import tpu_sc as plsc

_THRESHOLD = 0.01
_CHUNK = 1024


def _masked_gather(idx, weight, mask_real, b_per_w, n_chunks, num_cores):
    mesh = plsc.VectorSubcoreMesh(core_axis_name="c", subcore_axis_name="s")
    dim = weight.shape[1]

    @functools.partial(
        pl.kernel,
        mesh=mesh,
        compiler_params=pltpu.CompilerParams(use_tc_tiling_on_sc=False),
        out_type=jax.ShapeDtypeStruct((idx.shape[0], dim), jnp.float32),
        scratch_types=[
            pltpu.VMEM((_CHUNK,), jnp.int32),
            pltpu.VMEM((_CHUNK, dim), jnp.float32),
            pltpu.VMEM((_CHUNK, dim), jnp.float32),
            pltpu.SemaphoreType.DMA,
        ],
    )
    def k(idx_hbm, w_hbm, m_hbm, out_hbm, idx_v, w_v, m_v, sem):
        wid = lax.axis_index("s") * num_cores + lax.axis_index("c")
        base = wid * b_per_w

        def chunk_body(c, _):
            off = base + c * _CHUNK
            pltpu.sync_copy(idx_hbm.at[pl.ds(off, _CHUNK)], idx_v)
            cp_w = pltpu.async_copy(w_hbm.at[idx_v], w_v, sem)
            cp_m = pltpu.async_copy(m_hbm.at[idx_v], m_v, sem)
            cp_w.wait()
            cp_m.wait()

            def row_body(i, _):
                w = w_v[i, :]
                m = m_v[i, :]
                w_v[i, :] = jnp.where(m > _THRESHOLD, w, 0.0)
                return 0

            lax.fori_loop(0, _CHUNK, row_body, 0)
            pltpu.sync_copy(w_v, out_hbm.at[pl.ds(off, _CHUNK)])
            return 0

        lax.fori_loop(0, n_chunks, chunk_body, 0)

    return k(idx, weight, mask_real)


def kernel(x, weight, mask_real):
    idx = x.reshape(-1).astype(jnp.int32)
    b = idx.shape[0]
    info = plsc.get_sparse_core_info()
    nw = info.num_cores * info.num_subcores
    b_per_w = b // nw
    n_chunks = b_per_w // _CHUNK
    out = _masked_gather(idx, weight, mask_real, b_per_w, n_chunks,
                         info.num_cores)
    return out.reshape(x.shape + (weight.shape[1],))


# speedup vs baseline: 1.0909x; 1.0909x over previous
import functools

import jax
import jax.numpy as jnp
from jax import lax
from jax.experimental import pallas as pl
from jax.experimental.pallas import tpu as pltpu
from jax.experimental.pallas import tpu_sc as plsc

_THRESHOLD = 0.01
_CHUNK = 1024
_NBUF = 2


def _masked_gather(idx, weight, mask_real, b_per_w, n_chunks, num_cores):
    mesh = plsc.VectorSubcoreMesh(core_axis_name="c", subcore_axis_name="s")
    dim = weight.shape[1]

    @functools.partial(
        pl.kernel,
        mesh=mesh,
        compiler_params=pltpu.CompilerParams(use_tc_tiling_on_sc=False),
        out_type=jax.ShapeDtypeStruct((idx.shape[0], dim), jnp.float32),
        scratch_types=[
            pltpu.VMEM((b_per_w,), jnp.int32),
            [pltpu.VMEM((_CHUNK, dim), jnp.float32)] * _NBUF,
            [pltpu.VMEM((_CHUNK, dim), jnp.float32)] * _NBUF,
            [pltpu.VMEM((_CHUNK, dim), jnp.float32)] * _NBUF,
            [pltpu.SemaphoreType.DMA] * _NBUF,
            [pltpu.SemaphoreType.DMA] * _NBUF,
        ],
    )
    def k(idx_hbm, w_hbm, m_hbm, out_hbm, idx_v, w_v, m_v, o_v, sem_g,
          sem_o):
        wid = lax.axis_index("s") * num_cores + lax.axis_index("c")
        base = wid * b_per_w
        pltpu.sync_copy(idx_hbm.at[pl.ds(base, b_per_w)], idx_v)

        def fire_gathers(c, buf):
            ix = idx_v.at[pl.ds(c * _CHUNK, _CHUNK)]
            return (
                pltpu.async_copy(w_hbm.at[ix], w_v[buf], sem_g[buf]),
                pltpu.async_copy(m_hbm.at[ix], m_v[buf], sem_g[buf]),
            )

        pending_g = {}
        pending_o = {}
        for c in range(min(_NBUF, n_chunks)):
            pending_g[c] = fire_gathers(c, c % _NBUF)

        for c in range(n_chunks):
            buf = c % _NBUF
            for cp in pending_g.pop(c):
                cp.wait()
            if c - _NBUF in pending_o:
                pending_o.pop(c - _NBUF).wait()

            @plsc.parallel_loop(0, _CHUNK, unroll=8)
            def row_body(i):
                w = w_v[buf][i, :]
                m = m_v[buf][i, :]
                o_v[buf][i, :] = jnp.where(m > _THRESHOLD, w, 0.0)

            pending_o[c] = pltpu.async_copy(
                o_v[buf], out_hbm.at[pl.ds(base + c * _CHUNK, _CHUNK)],
                sem_o[buf])
            if c + _NBUF < n_chunks:
                pending_g[c + _NBUF] = fire_gathers(c + _NBUF, buf)

        for c in sorted(pending_o):
            pending_o[c].wait()

    return k(idx, weight, mask_real)


def kernel(x, weight, mask_real):
    idx = x.reshape(-1).astype(jnp.int32)
    b = idx.shape[0]
    info = plsc.get_sparse_core_info()
    nw = info.num_cores * info.num_subcores
    b_per_w = b // nw
    n_chunks = b_per_w // _CHUNK
    out = _masked_gather(idx, weight, mask_real, b_per_w, n_chunks,
                         info.num_cores)
    return out.reshape(x.shape + (weight.shape[1],))


# speedup vs baseline: 1.2136x; 1.1125x over previous
import functools

import jax
import jax.numpy as jnp
from jax import lax
from jax.experimental import pallas as pl
from jax.experimental.pallas import tpu as pltpu
from jax.experimental.pallas import tpu_sc as plsc

_THRESHOLD = 0.01
_NC = 2
_NW = 32
_FULL_BLOCKS = 7812
_X_UNITS = 896


def _phase_a(w_t, m_t, x_t, tail_rows):
    vocab = w_t.shape[1]
    dim = w_t.shape[0]
    n_b = x_t.shape[1]
    tail = vocab - _FULL_BLOCKS * 128
    mesh = plsc.VectorSubcoreMesh(core_axis_name="c", subcore_axis_name="s")

    @functools.partial(
        pl.kernel,
        mesh=mesh,
        compiler_params=pltpu.CompilerParams(use_tc_tiling_on_sc=True,
                                             needs_layout_passes=False,
                                             disable_bounds_checks=True),
        out_type=(
            jax.ShapeDtypeStruct((vocab * dim,), jnp.float32),
            jax.ShapeDtypeStruct((x_t.shape[0] * n_b,), jnp.int32),
        ),
        scratch_types=[
            pltpu.VMEM((dim, 128), jnp.float32),
            pltpu.VMEM((dim, 128), jnp.float32),
            pltpu.VMEM((128 * dim,), jnp.float32),
            pltpu.VMEM((8, 128), jnp.int32),
            pltpu.SemaphoreType.DMA,
        ],
    )
    def k(w_hbm, m_hbm, x_hbm, tl_hbm, tab_hbm, idx_hbm, wv, mv, ov, xv,
          sem):
        wid = lax.axis_index("s") * _NC + lax.axis_index("c")
        iota = lax.iota(jnp.int32, 16)

        def mask_cols(n_cols):
            def col(c, _):
                ci = jnp.full((16,), c, jnp.int32)
                w = plsc.load_gather(wv, [iota, ci])
                m = plsc.load_gather(mv, [iota, ci])
                ov[pl.ds(c * dim, dim)] = jnp.where(m > _THRESHOLD, w, 0.0)
                return 0

            lax.fori_loop(0, n_cols, col, 0)

        def vocab_block(kk, _):
            g = wid + kk * _NW

            @pl.when(g < _FULL_BLOCKS)
            def _():
                v0 = pl.multiple_of(g * 128, 128)
                cp_w = pltpu.async_copy(w_hbm.at[:, pl.ds(v0, 128)], wv, sem)
                cp_m = pltpu.async_copy(m_hbm.at[:, pl.ds(v0, 128)], mv, sem)
                cp_w.wait()
                cp_m.wait()
                mask_cols(128)
                pltpu.sync_copy(ov, tab_hbm.at[pl.ds(v0 * dim, 128 * dim)])

            @pl.when(g == _FULL_BLOCKS)
            def _():
                pltpu.sync_copy(tl_hbm, ov.at[pl.ds(0, tail * dim)])
                pltpu.sync_copy(
                    ov.at[pl.ds(0, tail * dim)],
                    tab_hbm.at[pl.ds(_FULL_BLOCKS * 128 * dim, tail * dim)])

            return 0

        lax.fori_loop(0, (_FULL_BLOCKS + 1 + _NW - 1) // _NW, vocab_block, 0)

        def x_unit(kk, _):
            u = wid + kk * _NW
            tj = u // 128
            tb = u % 128

            @pl.when(tj < 6)
            def _():
                ro = pl.multiple_of(tj * 8, 8)
                pltpu.sync_copy(
                    x_hbm.at[pl.ds(ro, 8), pl.ds(tb * 128, 128)], xv)
                for r in range(8):
                    j = tj * 8 + r
                    pltpu.sync_copy(
                        xv.at[r],
                        idx_hbm.at[pl.ds(j * n_b + tb * 128, 128)])

            @pl.when(tj == 6)
            def _():
                pltpu.sync_copy(
                    x_hbm.at[pl.ds(48, 2), pl.ds(tb * 128, 128)],
                    xv.at[pl.ds(0, 2)])
                for r in range(2):
                    pltpu.sync_copy(
                        xv.at[r],
                        idx_hbm.at[pl.ds((48 + r) * n_b + tb * 128, 128)])

            return 0

        lax.fori_loop(0, _X_UNITS // _NW, x_unit, 0)

    return k(w_t, m_t, x_t, tail_rows)


def _phase_b(idx_flat, tab, n_j, n_b):
    mesh = plsc.VectorSubcoreMesh(core_axis_name="c", subcore_axis_name="s")
    dim = tab.shape[1]
    n_units = n_j * (n_b // 128)

    @functools.partial(
        pl.kernel,
        mesh=mesh,
        compiler_params=pltpu.CompilerParams(use_tc_tiling_on_sc=False,
                                             needs_layout_passes=False),
        out_type=jax.ShapeDtypeStruct((n_j, 2, n_b // 128, 8, 128),
                                      jnp.float32),
        scratch_types=[
            pltpu.VMEM((128,), jnp.int32),
            pltpu.VMEM((128, dim), jnp.float32),
            pltpu.VMEM((dim, 128), jnp.float32),
            pltpu.SemaphoreType.DMA,
        ],
    )
    def k(idx_hbm, tab_hbm, out_hbm, iv, gv, tv, sem):
        wid = lax.axis_index("s") * _NC + lax.axis_index("c")
        iota = lax.iota(jnp.int32, 16)

        def unit(kk, _):
            u = wid + kk * _NW
            j = u // (n_b // 128)
            bb = u % (n_b // 128)
            pltpu.sync_copy(idx_hbm.at[pl.ds(j * n_b + bb * 128, 128)], iv)
            pltpu.async_copy(tab_hbm.at[iv], gv, sem).wait()

            def col(d, _):
                di = jnp.full((16,), d, jnp.int32)
                for k8 in range(8):
                    v = plsc.load_gather(gv, [iota + (k8 * 16), di])
                    tv[d, pl.ds(k8 * 16, 16)] = v
                return 0

            lax.fori_loop(0, dim, col, 0)
            pltpu.sync_copy(tv.at[pl.ds(0, 8)], out_hbm.at[j, 0, bb])
            pltpu.sync_copy(tv.at[pl.ds(8, 8)], out_hbm.at[j, 1, bb])
            return 0

        lax.fori_loop(0, n_units // _NW, unit, 0)

    return k(idx_flat, tab)


def kernel(x, weight, mask_real):
    n_b0, n_j = x.shape
    vocab, dim = weight.shape
    x_t = x.astype(jnp.int32).T
    v0 = _FULL_BLOCKS * 128
    tail_rows = ((mask_real[v0:] > _THRESHOLD) * weight[v0:]).reshape(-1)
    tab_flat, idx_flat = _phase_a(weight.T, mask_real.T, x_t, tail_rows)
    tab = tab_flat.reshape(vocab, dim)
    out5 = _phase_b(idx_flat, tab, n_j, n_b0)
    return out5.transpose(2, 4, 0, 1, 3).reshape(n_b0, n_j, dim)


# speedup vs baseline: 1.7887x; 1.4738x over previous
import functools

import jax
import jax.numpy as jnp
from jax import lax
from jax.experimental import pallas as pl
from jax.experimental.pallas import tpu as pltpu
from jax.experimental.pallas import tpu_sc as plsc

_THRESHOLD = 0.01
_NC = 2
_NW = 32
_BLK = 512
_NBLK = 1953
_ABLK = 62
_X_UNITS = 896
_BU = 512
_BUNITS = 50


def _phase_a(w_t, m_t, x_t, tail_rows):
    vocab = w_t.shape[1]
    dim = w_t.shape[0]
    n_b = x_t.shape[1]
    tail = vocab - _NBLK * _BLK
    mesh = plsc.VectorSubcoreMesh(core_axis_name="c", subcore_axis_name="s")

    @functools.partial(
        pl.kernel,
        mesh=mesh,
        compiler_params=pltpu.CompilerParams(use_tc_tiling_on_sc=True,
                                             needs_layout_passes=False),
        out_type=(
            jax.ShapeDtypeStruct((vocab * dim,), jnp.float32),
            jax.ShapeDtypeStruct((x_t.shape[0] * n_b,), jnp.int32),
        ),
        scratch_types=[
            pltpu.VMEM((2, dim, _BLK), jnp.float32),
            pltpu.VMEM((2, dim, _BLK), jnp.float32),
            pltpu.VMEM((2, _BLK * dim), jnp.float32),
            pltpu.VMEM((8, 128), jnp.int32),
            [pltpu.SemaphoreType.DMA] * 2,
            [pltpu.SemaphoreType.DMA] * 2,
            pltpu.SemaphoreType.DMA,
        ],
    )
    def k(w_hbm, m_hbm, x_hbm, tl_hbm, tab_hbm, idx_hbm, wv, mv, ov, xv,
          sem_in, sem_out, sem_x):
        wid = lax.axis_index("s") * _NC + lax.axis_index("c")
        iota = lax.iota(jnp.int32, 16)

        def blk_of(k_it):
            return pl.multiple_of(
                lax.rem(wid + k_it * _NW, _NBLK) * _BLK, _BLK)

        def fire_in(k_it, par):
            v0 = blk_of(k_it)
            pltpu.async_copy(w_hbm.at[:, pl.ds(v0, _BLK)], wv.at[par],
                             sem_in[par])
            pltpu.async_copy(m_hbm.at[:, pl.ds(v0, _BLK)], mv.at[par],
                             sem_in[par])

        def wait_in(k_it, par):
            v0 = blk_of(k_it)
            pltpu.make_async_copy(w_hbm.at[:, pl.ds(v0, _BLK)], wv.at[par],
                                  sem_in[par]).wait()
            pltpu.make_async_copy(m_hbm.at[:, pl.ds(v0, _BLK)], mv.at[par],
                                  sem_in[par]).wait()

        def wait_out(k_it, par):
            v0 = blk_of(k_it)
            pltpu.make_async_copy(
                ov.at[par], tab_hbm.at[pl.ds(v0 * dim, _BLK * dim)],
                sem_out[par]).wait()

        @pl.when(wid == 0)
        def _():
            pltpu.sync_copy(tl_hbm, ov.at[0, pl.ds(0, tail * dim)])
            pltpu.sync_copy(ov.at[0, pl.ds(0, tail * dim)],
                            tab_hbm.at[pl.ds(_NBLK * _BLK * dim,
                                             tail * dim)])

        fire_in(0, 0)
        fire_in(1, 1)

        def half_body(k_it, par):
            wait_in(k_it, par)

            @pl.when(k_it >= 2)
            def _():
                wait_out(k_it - 2, par)

            def col(c, _):
                ci = jnp.full((16,), c, jnp.int32)
                w = plsc.load_gather(wv.at[par], [iota, ci])
                m = plsc.load_gather(mv.at[par], [iota, ci])
                ov[par, pl.ds(c * dim, dim)] = jnp.where(m > _THRESHOLD, w,
                                                         0.0)
                return 0

            lax.fori_loop(0, _BLK, col, 0)
            v0 = blk_of(k_it)
            pltpu.async_copy(ov.at[par],
                             tab_hbm.at[pl.ds(v0 * dim, _BLK * dim)],
                             sem_out[par])

            @pl.when(k_it < _ABLK - 2)
            def _():
                fire_in(k_it + 2, par)

        def blk_pair(k2, _):
            half_body(k2 * 2, 0)
            half_body(k2 * 2 + 1, 1)
            return 0

        lax.fori_loop(0, _ABLK // 2, blk_pair, 0)
        wait_out(_ABLK - 2, 0)
        wait_out(_ABLK - 1, 1)

        def x_unit(kk, _):
            u = wid + kk * _NW
            tj = u // 128
            tb = u % 128

            def idx_dst(j):
                return idx_hbm.at[pl.ds(j * n_b + tb * 128, 128)]

            @pl.when(tj < 6)
            def _():
                ro = pl.multiple_of(tj * 8, 8)
                pltpu.sync_copy(
                    x_hbm.at[pl.ds(ro, 8), pl.ds(tb * 128, 128)], xv)
                for r in range(8):
                    pltpu.async_copy(xv.at[r], idx_dst(tj * 8 + r), sem_x)
                for r in range(8):
                    pltpu.make_async_copy(xv.at[r], idx_dst(tj * 8 + r),
                                          sem_x).wait()

            @pl.when(tj == 6)
            def _():
                pltpu.sync_copy(
                    x_hbm.at[pl.ds(48, 2), pl.ds(tb * 128, 128)],
                    xv.at[pl.ds(0, 2)])
                for r in range(2):
                    pltpu.async_copy(xv.at[r], idx_dst(48 + r), sem_x)
                for r in range(2):
                    pltpu.make_async_copy(xv.at[r], idx_dst(48 + r),
                                          sem_x).wait()

            return 0

        lax.fori_loop(0, _X_UNITS // _NW, x_unit, 0)

    return k(w_t, m_t, x_t, tail_rows)


def _phase_b(idx_flat, tab, n_j, n_b):
    mesh = plsc.VectorSubcoreMesh(core_axis_name="c", subcore_axis_name="s")
    dim = tab.shape[1]
    per_w = _BUNITS * _BU
    upj = n_b // _BU

    @functools.partial(
        pl.kernel,
        mesh=mesh,
        compiler_params=pltpu.CompilerParams(use_tc_tiling_on_sc=False,
                                             needs_layout_passes=False),
        out_type=jax.ShapeDtypeStruct((n_j, 2, n_b // 128, 8, 128),
                                      jnp.float32),
        scratch_types=[
            pltpu.VMEM((per_w,), jnp.int32),
            pltpu.VMEM((2, _BU, dim), jnp.float32),
            pltpu.VMEM((2, dim, _BU), jnp.float32),
            [pltpu.SemaphoreType.DMA] * 2,
            [pltpu.SemaphoreType.DMA] * 2,
        ],
    )
    def k(idx_hbm, tab_hbm, out_hbm, iv, gv, tv, sem_g, sem_o):
        wid = lax.axis_index("s") * _NC + lax.axis_index("c")
        iota = lax.iota(jnp.int32, 16)
        u0 = wid * _BUNITS

        pltpu.sync_copy(idx_hbm.at[pl.ds(wid * per_w, per_w)], iv)

        def fire_gather(u, par):
            pltpu.async_copy(tab_hbm.at[iv.at[pl.ds(u * _BU, _BU)]],
                             gv.at[par], sem_g[par])

        def wait_gather(u, par):
            pltpu.make_async_copy(tab_hbm.at[iv.at[pl.ds(u * _BU, _BU)]],
                                  gv.at[par], sem_g[par]).wait()

        def out_copies(u, par, fire):
            uu = u0 + u
            j = uu // upj
            q4 = lax.rem(uu, upj)
            for g in range(2):
                for cq in range(4):
                    src = tv.at[par, pl.ds(g * 8, 8), pl.ds(cq * 128, 128)]
                    dst = out_hbm.at[j, g, q4 * 4 + cq]
                    if fire:
                        pltpu.async_copy(src, dst, sem_o[par])
                    else:
                        pltpu.make_async_copy(src, dst, sem_o[par]).wait()

        fire_gather(0, 0)
        fire_gather(1, 1)

        def unit_half(u, par):
            wait_gather(u, par)

            @pl.when(u >= 2)
            def _():
                out_copies(u - 2, par, fire=False)

            def col(d, _):
                def seg(k8, _):
                    v = plsc.load_gather(gv.at[par],
                                         [iota + k8 * 16,
                                          jnp.full((16,), d, jnp.int32)])
                    tv[par, d, pl.ds(k8 * 16, 16)] = v
                    return 0

                lax.fori_loop(0, _BU // 16, seg, 0)
                return 0

            lax.fori_loop(0, dim, col, 0)
            out_copies(u, par, fire=True)

            @pl.when(u < _BUNITS - 2)
            def _():
                fire_gather(u + 2, par)

        def unit_pair(u2, _):
            unit_half(u2 * 2, 0)
            unit_half(u2 * 2 + 1, 1)
            return 0

        lax.fori_loop(0, _BUNITS // 2, unit_pair, 0)
        out_copies(_BUNITS - 2, 0, fire=False)
        out_copies(_BUNITS - 1, 1, fire=False)

    return k(idx_flat, tab)


def kernel(x, weight, mask_real):
    n_b0, n_j = x.shape
    vocab, dim = weight.shape
    x_t = x.astype(jnp.int32).T
    v0 = _NBLK * _BLK
    tail_rows = ((mask_real[v0:] > _THRESHOLD) * weight[v0:]).reshape(-1)
    tab_flat, idx_flat = _phase_a(weight.T, mask_real.T, x_t, tail_rows)
    tab = tab_flat.reshape(vocab, dim)
    out5 = _phase_b(idx_flat, tab, n_j, n_b0)
    return out5.transpose(2, 4, 0, 1, 3).reshape(n_b0, n_j, dim)


# speedup vs baseline: 3.1421x; 1.7567x over previous
import functools

import jax
import jax.numpy as jnp
from jax import lax
from jax.experimental import pallas as pl
from jax.experimental.pallas import tpu as pltpu
from jax.experimental.pallas import tpu_sc as plsc

_THRESHOLD = 0.01
_NC = 2
_NW = 32
_BLK = 512
_NBLK = 1953
_ABLK = 62
_X_UNITS = 896
_BU = 512
_BUNITS = 50


def _phase_a(w_t, m_t, x_t, tail_rows):
    vocab = w_t.shape[1]
    dim = w_t.shape[0]
    n_b = x_t.shape[1]
    tail = vocab - _NBLK * _BLK
    mesh = plsc.VectorSubcoreMesh(core_axis_name="c", subcore_axis_name="s")

    @functools.partial(
        pl.kernel,
        mesh=mesh,
        compiler_params=pltpu.CompilerParams(use_tc_tiling_on_sc=True,
                                             needs_layout_passes=False),
        out_type=(
            jax.ShapeDtypeStruct((vocab * dim,), jnp.float32),
            jax.ShapeDtypeStruct((x_t.shape[0] * n_b,), jnp.int32),
        ),
        scratch_types=[
            pltpu.VMEM((2, dim, _BLK + 1), jnp.float32),
            pltpu.VMEM((2, dim, _BLK + 1), jnp.float32),
            pltpu.VMEM((2, _BLK * dim), jnp.float32),
            pltpu.VMEM((8, 128), jnp.int32),
            [pltpu.SemaphoreType.DMA] * 2,
            [pltpu.SemaphoreType.DMA] * 2,
            pltpu.SemaphoreType.DMA,
        ],
    )
    def k(w_hbm, m_hbm, x_hbm, tl_hbm, tab_hbm, idx_hbm, wv, mv, ov, xv,
          sem_in, sem_out, sem_x):
        wid = lax.axis_index("s") * _NC + lax.axis_index("c")
        iota = lax.iota(jnp.int32, 16)

        def blk_of(k_it):
            return pl.multiple_of(
                lax.rem(wid + k_it * _NW, _NBLK) * _BLK, _BLK)

        def fire_in(k_it, par):
            v0 = blk_of(k_it)
            pltpu.async_copy(w_hbm.at[:, pl.ds(v0, _BLK)],
                             wv.at[par, :, pl.ds(0, _BLK)], sem_in[par])
            pltpu.async_copy(m_hbm.at[:, pl.ds(v0, _BLK)],
                             mv.at[par, :, pl.ds(0, _BLK)], sem_in[par])

        def wait_in(k_it, par):
            v0 = blk_of(k_it)
            pltpu.make_async_copy(w_hbm.at[:, pl.ds(v0, _BLK)],
                                  wv.at[par, :, pl.ds(0, _BLK)],
                                  sem_in[par]).wait()
            pltpu.make_async_copy(m_hbm.at[:, pl.ds(v0, _BLK)],
                                  mv.at[par, :, pl.ds(0, _BLK)],
                                  sem_in[par]).wait()

        def wait_out(k_it, par):
            v0 = blk_of(k_it)
            pltpu.make_async_copy(
                ov.at[par], tab_hbm.at[pl.ds(v0 * dim, _BLK * dim)],
                sem_out[par]).wait()

        @pl.when(wid == 0)
        def _():
            pltpu.sync_copy(tl_hbm, ov.at[0, pl.ds(0, tail * dim)])
            pltpu.sync_copy(ov.at[0, pl.ds(0, tail * dim)],
                            tab_hbm.at[pl.ds(_NBLK * _BLK * dim,
                                             tail * dim)])

        fire_in(0, 0)
        fire_in(1, 1)

        def half_body(k_it, par):
            wait_in(k_it, par)

            @pl.when(k_it >= 2)
            def _():
                wait_out(k_it - 2, par)

            @plsc.parallel_loop(0, _BLK, unroll=4)
            def col(c):
                ci = jnp.full((16,), c, jnp.int32)
                w = plsc.load_gather(wv.at[par], [iota, ci])
                m = plsc.load_gather(mv.at[par], [iota, ci])
                ov[par, pl.ds(c * dim, dim)] = jnp.where(m > _THRESHOLD, w,
                                                         0.0)
            v0 = blk_of(k_it)
            pltpu.async_copy(ov.at[par],
                             tab_hbm.at[pl.ds(v0 * dim, _BLK * dim)],
                             sem_out[par])

            @pl.when(k_it < _ABLK - 2)
            def _():
                fire_in(k_it + 2, par)

        def blk_pair(k2, _):
            half_body(k2 * 2, 0)
            half_body(k2 * 2 + 1, 1)
            return 0

        lax.fori_loop(0, _ABLK // 2, blk_pair, 0)
        wait_out(_ABLK - 2, 0)
        wait_out(_ABLK - 1, 1)

        def x_unit(kk, _):
            u = wid + kk * _NW
            tj = u // 128
            tb = u % 128

            def idx_dst(j):
                return idx_hbm.at[pl.ds(j * n_b + tb * 128, 128)]

            @pl.when(tj < 6)
            def _():
                ro = pl.multiple_of(tj * 8, 8)
                pltpu.sync_copy(
                    x_hbm.at[pl.ds(ro, 8), pl.ds(tb * 128, 128)], xv)
                for r in range(8):
                    pltpu.async_copy(xv.at[r], idx_dst(tj * 8 + r), sem_x)
                for r in range(8):
                    pltpu.make_async_copy(xv.at[r], idx_dst(tj * 8 + r),
                                          sem_x).wait()

            @pl.when(tj == 6)
            def _():
                pltpu.sync_copy(
                    x_hbm.at[pl.ds(48, 2), pl.ds(tb * 128, 128)],
                    xv.at[pl.ds(0, 2)])
                for r in range(2):
                    pltpu.async_copy(xv.at[r], idx_dst(48 + r), sem_x)
                for r in range(2):
                    pltpu.make_async_copy(xv.at[r], idx_dst(48 + r),
                                          sem_x).wait()

            return 0

        lax.fori_loop(0, _X_UNITS // _NW, x_unit, 0)

    return k(w_t, m_t, x_t, tail_rows)


def _phase_b(idx_flat, tab, n_j, n_b):
    mesh = plsc.VectorSubcoreMesh(core_axis_name="c", subcore_axis_name="s")
    dim = tab.shape[1]
    per_w = _BUNITS * _BU
    upj = n_b // _BU

    @functools.partial(
        pl.kernel,
        mesh=mesh,
        compiler_params=pltpu.CompilerParams(use_tc_tiling_on_sc=False,
                                             needs_layout_passes=False),
        out_type=jax.ShapeDtypeStruct((n_j, 2, n_b // 128, 8, 128),
                                      jnp.float32),
        scratch_types=[
            pltpu.VMEM((per_w,), jnp.int32),
            pltpu.VMEM((2, _BU, dim), jnp.float32),
            pltpu.VMEM((2, dim, _BU + 1), jnp.float32),
            [pltpu.SemaphoreType.DMA] * 2,
            [pltpu.SemaphoreType.DMA] * 2,
        ],
    )
    def k(idx_hbm, tab_hbm, out_hbm, iv, gv, tv, sem_g, sem_o):
        wid = lax.axis_index("s") * _NC + lax.axis_index("c")
        iota = lax.iota(jnp.int32, 16)
        u0 = wid * _BUNITS

        pltpu.sync_copy(idx_hbm.at[pl.ds(wid * per_w, per_w)], iv)

        def fire_gather(u, par):
            pltpu.async_copy(tab_hbm.at[iv.at[pl.ds(u * _BU, _BU)]],
                             gv.at[par], sem_g[par])

        def wait_gather(u, par):
            pltpu.make_async_copy(tab_hbm.at[iv.at[pl.ds(u * _BU, _BU)]],
                                  gv.at[par], sem_g[par]).wait()

        def out_copies(u, par, fire):
            uu = u0 + u
            j = uu // upj
            q4 = lax.rem(uu, upj)
            for g in range(2):
                for cq in range(4):
                    src = tv.at[par, pl.ds(g * 8, 8), pl.ds(cq * 128, 128)]
                    dst = out_hbm.at[j, g, q4 * 4 + cq]
                    if fire:
                        pltpu.async_copy(src, dst, sem_o[par])
                    else:
                        pltpu.make_async_copy(src, dst, sem_o[par]).wait()

        fire_gather(0, 0)
        fire_gather(1, 1)

        def unit_half(u, par):
            wait_gather(u, par)

            @pl.when(u >= 2)
            def _():
                out_copies(u - 2, par, fire=False)

            @plsc.parallel_loop(0, _BU, unroll=4)
            def row(b):
                v = gv[par, b, :]
                plsc.store_scatter(tv.at[par],
                                   [iota, jnp.full((16,), b, jnp.int32)], v)

            out_copies(u, par, fire=True)

            @pl.when(u < _BUNITS - 2)
            def _():
                fire_gather(u + 2, par)

        def unit_pair(u2, _):
            unit_half(u2 * 2, 0)
            unit_half(u2 * 2 + 1, 1)
            return 0

        lax.fori_loop(0, _BUNITS // 2, unit_pair, 0)
        out_copies(_BUNITS - 2, 0, fire=False)
        out_copies(_BUNITS - 1, 1, fire=False)

    return k(idx_flat, tab)


def kernel(x, weight, mask_real):
    n_b0, n_j = x.shape
    vocab, dim = weight.shape
    x_t = x.astype(jnp.int32).T
    v0 = _NBLK * _BLK
    tail_rows = ((mask_real[v0:] > _THRESHOLD) * weight[v0:]).reshape(-1)
    tab_flat, idx_flat = _phase_a(weight.T, mask_real.T, x_t, tail_rows)
    tab = tab_flat.reshape(vocab, dim)
    out5 = _phase_b(idx_flat, tab, n_j, n_b0)
    return out5.transpose(2, 4, 0, 1, 3).reshape(n_b0, n_j, dim)


# speedup vs baseline: 3.1433x; 1.0004x over previous
import functools

import jax
import jax.numpy as jnp
from jax import lax
from jax.experimental import pallas as pl
from jax.experimental.pallas import tpu as pltpu
from jax.experimental.pallas import tpu_sc as plsc

_THRESHOLD = 0.01
_NC = 2
_NW = 32
_BLK = 512
_NBLK = 1953
_ABLK = 62
_X_UNITS = 896
_BU = 512
_BUNITS = 50


def _phase_a(w_t, m_t, tail_rows):
    vocab = w_t.shape[1]
    dim = w_t.shape[0]
    tail = vocab - _NBLK * _BLK
    mesh = plsc.VectorSubcoreMesh(core_axis_name="c", subcore_axis_name="s")

    @functools.partial(
        pl.kernel,
        mesh=mesh,
        compiler_params=pltpu.CompilerParams(use_tc_tiling_on_sc=True,
                                             needs_layout_passes=False),
        out_type=jax.ShapeDtypeStruct((vocab * dim,), jnp.float32),
        scratch_types=[
            pltpu.VMEM((2, dim, _BLK + 1), jnp.float32),
            pltpu.VMEM((2, dim, _BLK + 1), jnp.float32),
            pltpu.VMEM((2, _BLK * dim), jnp.float32),
            [pltpu.SemaphoreType.DMA] * 2,
            [pltpu.SemaphoreType.DMA] * 2,
        ],
    )
    def k(w_hbm, m_hbm, tl_hbm, tab_hbm, wv, mv, ov, sem_in, sem_out):
        wid = lax.axis_index("s") * _NC + lax.axis_index("c")
        iota = lax.iota(jnp.int32, 16)

        def blk_of(k_it):
            return pl.multiple_of(
                lax.rem(wid + k_it * _NW, _NBLK) * _BLK, _BLK)

        def fire_in(k_it, par):
            v0 = blk_of(k_it)
            pltpu.async_copy(w_hbm.at[:, pl.ds(v0, _BLK)],
                             wv.at[par, :, pl.ds(0, _BLK)], sem_in[par])
            pltpu.async_copy(m_hbm.at[:, pl.ds(v0, _BLK)],
                             mv.at[par, :, pl.ds(0, _BLK)], sem_in[par])

        def wait_in(k_it, par):
            v0 = blk_of(k_it)
            pltpu.make_async_copy(w_hbm.at[:, pl.ds(v0, _BLK)],
                                  wv.at[par, :, pl.ds(0, _BLK)],
                                  sem_in[par]).wait()
            pltpu.make_async_copy(m_hbm.at[:, pl.ds(v0, _BLK)],
                                  mv.at[par, :, pl.ds(0, _BLK)],
                                  sem_in[par]).wait()

        def wait_out(k_it, par):
            v0 = blk_of(k_it)
            pltpu.make_async_copy(
                ov.at[par], tab_hbm.at[pl.ds(v0 * dim, _BLK * dim)],
                sem_out[par]).wait()

        @pl.when(wid == 0)
        def _():
            pltpu.sync_copy(tl_hbm, ov.at[0, pl.ds(0, tail * dim)])
            pltpu.sync_copy(ov.at[0, pl.ds(0, tail * dim)],
                            tab_hbm.at[pl.ds(_NBLK * _BLK * dim,
                                             tail * dim)])

        fire_in(0, 0)
        fire_in(1, 1)

        def half_body(k_it, par):
            wait_in(k_it, par)

            @pl.when(k_it >= 2)
            def _():
                wait_out(k_it - 2, par)

            @plsc.parallel_loop(0, _BLK, unroll=8)
            def col(c):
                ci = jnp.full((16,), c, jnp.int32)
                w = plsc.load_gather(wv.at[par], [iota, ci])
                m = plsc.load_gather(mv.at[par], [iota, ci])
                ov[par, pl.ds(c * dim, dim)] = jnp.where(m > _THRESHOLD, w,
                                                         0.0)
            v0 = blk_of(k_it)
            pltpu.async_copy(ov.at[par],
                             tab_hbm.at[pl.ds(v0 * dim, _BLK * dim)],
                             sem_out[par])

            @pl.when(k_it < _ABLK - 2)
            def _():
                fire_in(k_it + 2, par)

        def blk_pair(k2, _):
            half_body(k2 * 2, 0)
            half_body(k2 * 2 + 1, 1)
            return 0

        lax.fori_loop(0, _ABLK // 2, blk_pair, 0)
        wait_out(_ABLK - 2, 0)
        wait_out(_ABLK - 1, 1)

    return k(w_t, m_t, tail_rows)


def _phase_b(idx_flat, tab, n_j, n_b):
    mesh = plsc.VectorSubcoreMesh(core_axis_name="c", subcore_axis_name="s")
    dim = tab.shape[1]
    per_w = _BUNITS * _BU
    upj = n_b // _BU

    @functools.partial(
        pl.kernel,
        mesh=mesh,
        compiler_params=pltpu.CompilerParams(use_tc_tiling_on_sc=False,
                                             needs_layout_passes=False),
        out_type=jax.ShapeDtypeStruct((n_j, 2, n_b // 128, 8, 128),
                                      jnp.float32),
        scratch_types=[
            pltpu.VMEM((per_w,), jnp.int32),
            pltpu.VMEM((2, _BU, dim), jnp.float32),
            pltpu.VMEM((2, dim, _BU + 1), jnp.float32),
            [pltpu.SemaphoreType.DMA] * 2,
            [pltpu.SemaphoreType.DMA] * 2,
        ],
    )
    def k(idx_hbm, tab_hbm, out_hbm, iv, gv, tv, sem_g, sem_o):
        wid = lax.axis_index("s") * _NC + lax.axis_index("c")
        iota = lax.iota(jnp.int32, 16)
        u0 = wid * _BUNITS

        pltpu.sync_copy(idx_hbm.at[pl.ds(wid * per_w, per_w)], iv)

        def fire_gather(u, par):
            pltpu.async_copy(tab_hbm.at[iv.at[pl.ds(u * _BU, _BU)]],
                             gv.at[par], sem_g[par])

        def wait_gather(u, par):
            pltpu.make_async_copy(tab_hbm.at[iv.at[pl.ds(u * _BU, _BU)]],
                                  gv.at[par], sem_g[par]).wait()

        def out_copies(u, par, fire):
            uu = u0 + u
            j = uu // upj
            q4 = lax.rem(uu, upj)
            for g in range(2):
                for cq in range(4):
                    src = tv.at[par, pl.ds(g * 8, 8), pl.ds(cq * 128, 128)]
                    dst = out_hbm.at[j, g, q4 * 4 + cq]
                    if fire:
                        pltpu.async_copy(src, dst, sem_o[par])
                    else:
                        pltpu.make_async_copy(src, dst, sem_o[par]).wait()

        fire_gather(0, 0)
        fire_gather(1, 1)

        def unit_half(u, par):
            wait_gather(u, par)

            @pl.when(u >= 2)
            def _():
                out_copies(u - 2, par, fire=False)

            @plsc.parallel_loop(0, _BU, unroll=4)
            def row(b):
                v = gv[par, b, :]
                plsc.store_scatter(tv.at[par],
                                   [iota, jnp.full((16,), b, jnp.int32)], v)

            out_copies(u, par, fire=True)

            @pl.when(u < _BUNITS - 2)
            def _():
                fire_gather(u + 2, par)

        def unit_pair(u2, _):
            unit_half(u2 * 2, 0)
            unit_half(u2 * 2 + 1, 1)
            return 0

        lax.fori_loop(0, _BUNITS // 2, unit_pair, 0)
        out_copies(_BUNITS - 2, 0, fire=False)
        out_copies(_BUNITS - 1, 1, fire=False)

    return k(idx_flat, tab)


def kernel(x, weight, mask_real):
    n_b0, n_j = x.shape
    vocab, dim = weight.shape
    x_t = x.astype(jnp.int32).T
    v0 = _NBLK * _BLK
    tail_rows = ((mask_real[v0:] > _THRESHOLD) * weight[v0:]).reshape(-1)
    idx_flat = x_t.reshape(-1)
    tab_flat = _phase_a(weight.T, mask_real.T, tail_rows)
    tab = tab_flat.reshape(vocab, dim)
    out5 = _phase_b(idx_flat, tab, n_j, n_b0)
    return out5.transpose(2, 4, 0, 1, 3).reshape(n_b0, n_j, dim)


# speedup vs baseline: 4.6410x; 1.4765x over previous
import functools

import jax
import jax.numpy as jnp
from jax import lax
from jax.experimental import pallas as pl
from jax.experimental.pallas import tpu as pltpu
from jax.experimental.pallas import tpu_sc as plsc

_THRESHOLD = 0.01
_NC = 2
_NW = 32
_BLK = 512
_NBLK = 1953
_ABLK = 62
_X_UNITS = 896
_BU = 512
_BUNITS = 50


def _phase_a(w_t, m_t, tail_rows):
    vocab = w_t.shape[1]
    dim = w_t.shape[0]
    tail = vocab - _NBLK * _BLK
    mesh = plsc.VectorSubcoreMesh(core_axis_name="c", subcore_axis_name="s")

    @functools.partial(
        pl.kernel,
        mesh=mesh,
        compiler_params=pltpu.CompilerParams(use_tc_tiling_on_sc=True,
                                             needs_layout_passes=False),
        out_type=jax.ShapeDtypeStruct((vocab * dim,), jnp.float32),
        scratch_types=[
            pltpu.VMEM((2, dim, _BLK + 1), jnp.float32),
            pltpu.VMEM((2, dim, _BLK + 1), jnp.float32),
            pltpu.VMEM((2, _BLK * dim), jnp.float32),
            [pltpu.SemaphoreType.DMA] * 2,
            [pltpu.SemaphoreType.DMA] * 2,
        ],
    )
    def k(w_hbm, m_hbm, tl_hbm, tab_hbm, wv, mv, ov, sem_in, sem_out):
        wid = lax.axis_index("s") * _NC + lax.axis_index("c")
        iota = lax.iota(jnp.int32, 16)

        def blk_of(k_it):
            return pl.multiple_of(
                lax.rem(wid + k_it * _NW, _NBLK) * _BLK, _BLK)

        def fire_in(k_it, par):
            v0 = blk_of(k_it)
            pltpu.async_copy(w_hbm.at[:, pl.ds(v0, _BLK)],
                             wv.at[par, :, pl.ds(0, _BLK)], sem_in[par])
            pltpu.async_copy(m_hbm.at[:, pl.ds(v0, _BLK)],
                             mv.at[par, :, pl.ds(0, _BLK)], sem_in[par])

        def wait_in(k_it, par):
            v0 = blk_of(k_it)
            pltpu.make_async_copy(w_hbm.at[:, pl.ds(v0, _BLK)],
                                  wv.at[par, :, pl.ds(0, _BLK)],
                                  sem_in[par]).wait()
            pltpu.make_async_copy(m_hbm.at[:, pl.ds(v0, _BLK)],
                                  mv.at[par, :, pl.ds(0, _BLK)],
                                  sem_in[par]).wait()

        def wait_out(k_it, par):
            v0 = blk_of(k_it)
            pltpu.make_async_copy(
                ov.at[par], tab_hbm.at[pl.ds(v0 * dim, _BLK * dim)],
                sem_out[par]).wait()

        @pl.when(wid == 0)
        def _():
            pltpu.sync_copy(tl_hbm, ov.at[0, pl.ds(0, tail * dim)])
            pltpu.sync_copy(ov.at[0, pl.ds(0, tail * dim)],
                            tab_hbm.at[pl.ds(_NBLK * _BLK * dim,
                                             tail * dim)])

        fire_in(0, 0)
        fire_in(1, 1)

        def half_body(k_it, par):
            wait_in(k_it, par)

            @pl.when(k_it >= 2)
            def _():
                wait_out(k_it - 2, par)

            def mrow(r, _):
                @plsc.parallel_loop(0, _BLK // 16, unroll=8)
                def seg(s):
                    off = s * 16
                    w = wv[par, r, pl.ds(off, 16)]
                    m = mv[par, r, pl.ds(off, 16)]
                    wv[par, r, pl.ds(off, 16)] = jnp.where(
                        m > _THRESHOLD, w, 0.0)

                return 0

            lax.fori_loop(0, dim, mrow, 0)

            @plsc.parallel_loop(0, _BLK, unroll=16)
            def col(c):
                ci = jnp.full((16,), c, jnp.int32)
                ov[par, pl.ds(c * dim, dim)] = plsc.load_gather(
                    wv.at[par], [iota, ci])
            v0 = blk_of(k_it)
            pltpu.async_copy(ov.at[par],
                             tab_hbm.at[pl.ds(v0 * dim, _BLK * dim)],
                             sem_out[par])

            @pl.when(k_it < _ABLK - 2)
            def _():
                fire_in(k_it + 2, par)

        def blk_pair(k2, _):
            half_body(k2 * 2, 0)
            half_body(k2 * 2 + 1, 1)
            return 0

        lax.fori_loop(0, _ABLK // 2, blk_pair, 0)
        wait_out(_ABLK - 2, 0)
        wait_out(_ABLK - 1, 1)

    return k(w_t, m_t, tail_rows)


def _phase_b(idx_flat, tab, n_j, n_b):
    mesh = plsc.VectorSubcoreMesh(core_axis_name="c", subcore_axis_name="s")
    dim = tab.shape[1]
    per_w = _BUNITS * _BU
    upj = n_b // _BU

    @functools.partial(
        pl.kernel,
        mesh=mesh,
        compiler_params=pltpu.CompilerParams(use_tc_tiling_on_sc=False,
                                             needs_layout_passes=False),
        out_type=jax.ShapeDtypeStruct((n_j, 2, n_b // 128, 8, 128),
                                      jnp.float32),
        scratch_types=[
            pltpu.VMEM((per_w,), jnp.int32),
            pltpu.VMEM((2, _BU, dim), jnp.float32),
            pltpu.VMEM((2, dim, _BU + 1), jnp.float32),
            [pltpu.SemaphoreType.DMA] * 2,
            [pltpu.SemaphoreType.DMA] * 2,
        ],
    )
    def k(idx_hbm, tab_hbm, out_hbm, iv, gv, tv, sem_g, sem_o):
        wid = lax.axis_index("s") * _NC + lax.axis_index("c")
        iota = lax.iota(jnp.int32, 16)
        u0 = wid * _BUNITS

        pltpu.sync_copy(idx_hbm.at[pl.ds(wid * per_w, per_w)], iv)

        def fire_gather(u, par):
            pltpu.async_copy(tab_hbm.at[iv.at[pl.ds(u * _BU, _BU)]],
                             gv.at[par], sem_g[par])

        def wait_gather(u, par):
            pltpu.make_async_copy(tab_hbm.at[iv.at[pl.ds(u * _BU, _BU)]],
                                  gv.at[par], sem_g[par]).wait()

        def out_copies(u, par, fire):
            uu = u0 + u
            j = uu // upj
            q4 = lax.rem(uu, upj)
            for g in range(2):
                for cq in range(4):
                    src = tv.at[par, pl.ds(g * 8, 8), pl.ds(cq * 128, 128)]
                    dst = out_hbm.at[j, g, q4 * 4 + cq]
                    if fire:
                        pltpu.async_copy(src, dst, sem_o[par])
                    else:
                        pltpu.make_async_copy(src, dst, sem_o[par]).wait()

        fire_gather(0, 0)
        fire_gather(1, 1)

        def unit_half(u, par):
            wait_gather(u, par)

            @pl.when(u >= 2)
            def _():
                out_copies(u - 2, par, fire=False)

            @plsc.parallel_loop(0, _BU, unroll=4)
            def row(b):
                v = gv[par, b, :]
                plsc.store_scatter(tv.at[par],
                                   [iota, jnp.full((16,), b, jnp.int32)], v)

            out_copies(u, par, fire=True)

            @pl.when(u < _BUNITS - 2)
            def _():
                fire_gather(u + 2, par)

        def unit_pair(u2, _):
            unit_half(u2 * 2, 0)
            unit_half(u2 * 2 + 1, 1)
            return 0

        lax.fori_loop(0, _BUNITS // 2, unit_pair, 0)
        out_copies(_BUNITS - 2, 0, fire=False)
        out_copies(_BUNITS - 1, 1, fire=False)

    return k(idx_flat, tab)


def kernel(x, weight, mask_real):
    n_b0, n_j = x.shape
    vocab, dim = weight.shape
    x_t = x.astype(jnp.int32).T
    v0 = _NBLK * _BLK
    tail_rows = ((mask_real[v0:] > _THRESHOLD) * weight[v0:]).reshape(-1)
    idx_flat = x_t.reshape(-1)
    tab_flat = _phase_a(weight.T, mask_real.T, tail_rows)
    tab = tab_flat.reshape(vocab, dim)
    out5 = _phase_b(idx_flat, tab, n_j, n_b0)
    return out5.transpose(2, 4, 0, 1, 3).reshape(n_b0, n_j, dim)
